# Initial kernel scaffold; baseline (speedup 1.0000x reference)
import jax
import jax.numpy as jnp
from jax.experimental import pallas as pl


def kernel(predictions, targets, cost_matrix):
    raise NotImplementedError("write your pallas kernel here")


# speedup vs baseline: 1.0928x; 1.0928x over previous
import functools

import jax
import jax.numpy as jnp
from jax import lax
from jax.experimental import pallas as pl
from jax.experimental.pallas import tpu as pltpu
from jax.experimental.pallas import tpu_sc as plsc

B = 16384
C = 1000
CP = 1024

_NC = 2
_NS = 16
_NW = _NC * _NS
_B_PER_W = B // _NW
_CHUNK = 64
_N_CHUNKS = _B_PER_W // _CHUNK


def _sc_gather(table, idx):
    mesh = plsc.VectorSubcoreMesh(core_axis_name="c", subcore_axis_name="s")

    @functools.partial(
        pl.kernel,
        mesh=mesh,
        out_type=jax.ShapeDtypeStruct((B, CP), jnp.float32),
        scratch_types=[
            pltpu.VMEM((_CHUNK,), jnp.int32),
            pltpu.VMEM((_CHUNK, CP), jnp.float32),
            pltpu.SemaphoreType.DMA,
        ],
    )
    def k(table_hbm, idx_hbm, out_hbm, idx_v, rows_v, sem):
        wid = lax.axis_index("s") * _NC + lax.axis_index("c")
        base = wid * _B_PER_W

        def body(j, _):
            off = base + j * _CHUNK
            pltpu.sync_copy(idx_hbm.at[pl.ds(off, _CHUNK)], idx_v)
            pltpu.async_copy(table_hbm.at[idx_v], rows_v, sem).wait()
            pltpu.sync_copy(rows_v, out_hbm.at[pl.ds(off, _CHUNK)])
            return ()

        lax.fori_loop(0, _N_CHUNKS, body, (), unroll=False)

    return k(table, idx)


_BLK = 512


def _tc_body(pred_ref, rows_ref, out_ref):
    x = pred_ref[...]
    rw = rows_ref[...]
    m = jnp.max(x, axis=-1, keepdims=True)
    e = jnp.exp(x - m)
    z = jnp.sum(e, axis=-1, keepdims=True)
    ep = jnp.pad(e, ((0, 0), (0, CP - C)))
    pred_num = jnp.sum(rw * ep, axis=-1, keepdims=True)
    lane = lax.broadcasted_iota(jnp.int32, (_BLK, CP), 1)
    opt = jnp.sum(jnp.where(lane == C, rw, 0.0), axis=-1, keepdims=True)
    blk_sum = jnp.sum(pred_num / z - opt)

    @pl.when(pl.program_id(0) == 0)
    def _():
        out_ref[...] = jnp.zeros_like(out_ref)

    out_ref[...] += blk_sum


def _tc_loss(predictions, rows):
    grid = (B // _BLK,)
    out = pl.pallas_call(
        _tc_body,
        grid=grid,
        in_specs=[
            pl.BlockSpec((_BLK, C), lambda i: (i, 0)),
            pl.BlockSpec((_BLK, CP), lambda i: (i, 0)),
        ],
        out_specs=pl.BlockSpec((1, 1), lambda i: (0, 0)),
        out_shape=jax.ShapeDtypeStruct((1, 1), jnp.float32),
    )(predictions, rows)
    return out[0, 0]


def kernel(predictions, targets, cost_matrix):
    tgt = targets.astype(jnp.int32)
    diag = jnp.diagonal(cost_matrix)
    table = jnp.zeros((CP, CP), jnp.float32)
    table = table.at[:C, :C].set(cost_matrix)
    table = table.at[:C, C].set(diag)
    rows = _sc_gather(table, tgt)
    total = _tc_loss(predictions, rows)
    return total / jnp.float32(B)
